# baseline (speedup 1.0000x reference)
import functools

import jax
import jax.numpy as jnp
from jax.experimental import pallas as pl
from jax.experimental.pallas import tpu as pltpu

B = 2
N = 256
DIM = 512
H = 8
DH = 64
CH = 16
REL_DIST_SCALE = 100.0
REL_DIST_CUTOFF = 5000.0
ROTARY_THETA = 10000.0
SCALE = DH ** (-0.5)

BI = 32
NI = N // BI
HP = DH // 2
HD2 = DIM // 2

_PREC = jax.lax.Precision.HIGHEST


def _gelu(x):
    return x * (jax.lax.erf(x * (2.0 ** -0.5)) + 1.0) * 0.5


def _softmax_lanes(x):
    m = jnp.max(x, axis=-1, keepdims=True)
    e = jnp.exp(x - m)
    return e / jnp.sum(e, axis=-1, keepdims=True)


def _evenodd_rot(xT):
    x2 = xT.reshape(HD2, 2, N)
    te = x2[:, 0, :]
    to = x2[:, 1, :]
    x4 = xT.reshape(DIM // 4, 4, N)
    x0 = x4[:, 0, :].reshape(H, CH, N)
    x1 = x4[:, 1, :].reshape(H, CH, N)
    x2b = x4[:, 2, :].reshape(H, CH, N)
    x3 = x4[:, 3, :].reshape(H, CH, N)
    ra = jnp.concatenate([-x1, x0], axis=1).reshape(HD2, N)
    rb = jnp.concatenate([-x3, x2b], axis=1).reshape(HD2, N)
    return te, to, ra, rb


def _body(feats_ref, coors_ref, coorsT_ref, Wq_ref, WkvT_ref, Wout_ref,
          bout_ref, invf_ref, Wc1_ref, bc1_ref, Wc2_ref, bc2_ref,
          Wg_ref, bg_ref, comb_ref, cns_ref,
          out_ref, cout_ref,
          qs_ref, kTe_ref, kTo_ref, kA_ref, kB_ref,
          vTe_ref, vTo_ref, vA_ref, vB_ref):
    i = pl.program_id(1)

    @pl.when(i == 0)
    def _setup():
        f = feats_ref[0]
        qs_ref[...] = jnp.dot(f, Wq_ref[...],
                              preferred_element_type=jnp.float32,
                              precision=_PREC) * SCALE
        kvT = jnp.dot(WkvT_ref[...], f.T,
                      preferred_element_type=jnp.float32,
                      precision=_PREC)
        kTe_ref[...], kTo_ref[...], kA_ref[...], kB_ref[...] = (
            _evenodd_rot(kvT[:DIM]))
        vTe_ref[...], vTo_ref[...], vA_ref[...], vB_ref[...] = (
            _evenodd_rot(kvT[DIM:]))

    ci = coors_ref[0, pl.ds(i * BI, BI), :]
    cj = coorsT_ref[0]
    rel = [ci[:, c:c + 1] - cj[c:c + 1, :] for c in range(3)]
    sq = rel[0] * rel[0] + rel[1] * rel[1] + rel[2] * rel[2]
    dist = jnp.where(sq == 0.0, 0.0, jnp.sqrt(jnp.where(sq == 0.0, 1.0, sq)))
    rd = jnp.minimum(dist * REL_DIST_SCALE, REL_DIST_CUTOFF)
    f3 = rd[:, None, :] * invf_ref[...][None, :, :]
    cosf = jnp.cos(f3)
    sinf = jnp.sin(f3)

    q = qs_ref[pl.ds(i * BI, BI), :]
    qeo = q.reshape(BI, H, HP, 2)
    qe = qeo[:, :, :, 0].transpose(1, 0, 2)[:, :, :, None]
    qo = qeo[:, :, :, 1].transpose(1, 0, 2)[:, :, :, None]
    kTe4 = kTe_ref[...].reshape(H, 1, HP, N)
    kTo4 = kTo_ref[...].reshape(H, 1, HP, N)
    kA4 = kA_ref[...].reshape(H, 1, HP, N)
    kB4 = kB_ref[...].reshape(H, 1, HP, N)
    cos4 = cosf[None]
    sin4 = sinf[None]
    c4 = qe * kTe4 + qo * kTo4
    s4 = qe * kA4 + qo * kB4
    t4 = c4 * cos4 + s4 * sin4
    qk = jnp.sum(t4, axis=2)

    hmid = []
    for c in range(CH):
        s = qk[0] * Wc1_ref[0, c]
        for h in range(1, H):
            s = s + qk[h] * Wc1_ref[h, c]
        hmid.append(_gelu(s + bc1_ref[0, c]))
    wsum = None
    for hh in range(H):
        cw = hmid[0] * Wc2_ref[0, hh]
        for c in range(1, CH):
            cw = cw + hmid[c] * Wc2_ref[c, hh]
        cw = cw + bc2_ref[0, hh]
        g = qk[0] * Wg_ref[0, hh]
        for h in range(1, H):
            g = g + qk[h] * Wg_ref[h, hh]
        g = jnp.tanh(g + bg_ref[0, hh])
        term = _softmax_lanes(cw) * g * comb_ref[0, hh]
        wsum = term if wsum is None else wsum + term
    inv = cns_ref[0, 0] / jnp.maximum(dist, 1e-8)
    wr = wsum * inv
    cout_ref[0] = jnp.concatenate(
        [jnp.sum(wr * rel[c], axis=1, keepdims=True) for c in range(3)],
        axis=1)

    attn = _softmax_lanes(qk)
    ac4 = attn[:, :, None, :] * cosf[None]
    as4 = attn[:, :, None, :] * sinf[None]
    vTe4 = vTe_ref[...].reshape(H, 1, HP, N)
    vTo4 = vTo_ref[...].reshape(H, 1, HP, N)
    vA4 = vA_ref[...].reshape(H, 1, HP, N)
    vB4 = vB_ref[...].reshape(H, 1, HP, N)
    oe = jnp.sum(ac4 * vTe4 + as4 * vA4, axis=3)
    oo = jnp.sum(ac4 * vTo4 + as4 * vB4, axis=3)
    out_cat = jnp.stack([oe, oo], axis=-1).transpose(1, 0, 2, 3)
    out_cat = out_cat.reshape(BI, DIM)
    out_ref[0, pl.ds(i * BI, BI), :] = out_cat

    @pl.when(i == NI - 1)
    def _project():
        out_ref[0] = jnp.dot(out_ref[0], Wout_ref[...],
                             preferred_element_type=jnp.float32,
                             precision=_PREC) + bout_ref[...]


@jax.jit
def kernel(feats, coors, W_qkv, W_out, b_out, W_c1, b_c1, W_c2, b_c2,
           W_gate, b_gate, coors_norm_scale, coors_combine):
    W_q = W_qkv[:, :DIM]
    W_kvT = W_qkv[:, DIM:].T
    coorsT = coors.transpose(0, 2, 1)
    invf = (1.0 / (ROTARY_THETA **
                   (jnp.arange(0, DH, 2, dtype=jnp.float32) / DH)))
    invf = jnp.tile(invf.reshape(HP, 1), (1, N))
    vfull = lambda shape, imap: pl.BlockSpec(shape, imap)
    const2 = lambda b, i: (0, 0)
    out, cout = pl.pallas_call(
        _body,
        grid=(B, NI),
        in_specs=[
            vfull((1, N, DIM), lambda b, i: (b, 0, 0)),
            vfull((1, N, 3), lambda b, i: (b, 0, 0)),
            vfull((1, 3, N), lambda b, i: (b, 0, 0)),
            vfull((DIM, DIM), const2),
            vfull((2 * DIM, DIM), const2),
            vfull((DIM, DIM), const2),
            vfull((1, DIM), const2),
            vfull((HP, N), const2),
            pl.BlockSpec(memory_space=pltpu.SMEM),
            pl.BlockSpec(memory_space=pltpu.SMEM),
            pl.BlockSpec(memory_space=pltpu.SMEM),
            pl.BlockSpec(memory_space=pltpu.SMEM),
            pl.BlockSpec(memory_space=pltpu.SMEM),
            pl.BlockSpec(memory_space=pltpu.SMEM),
            pl.BlockSpec(memory_space=pltpu.SMEM),
            pl.BlockSpec(memory_space=pltpu.SMEM),
        ],
        out_specs=[
            pl.BlockSpec((1, N, DIM), lambda b, i: (b, 0, 0)),
            pl.BlockSpec((1, BI, 3), lambda b, i: (b, i, 0)),
        ],
        out_shape=[
            jax.ShapeDtypeStruct((B, N, DIM), jnp.float32),
            jax.ShapeDtypeStruct((B, N, 3), jnp.float32),
        ],
        scratch_shapes=[
            pltpu.VMEM((N, DIM), jnp.float32),
            pltpu.VMEM((HD2, N), jnp.float32),
            pltpu.VMEM((HD2, N), jnp.float32),
            pltpu.VMEM((HD2, N), jnp.float32),
            pltpu.VMEM((HD2, N), jnp.float32),
            pltpu.VMEM((HD2, N), jnp.float32),
            pltpu.VMEM((HD2, N), jnp.float32),
            pltpu.VMEM((HD2, N), jnp.float32),
            pltpu.VMEM((HD2, N), jnp.float32),
        ],
        compiler_params=pltpu.CompilerParams(
            dimension_semantics=("arbitrary", "arbitrary"),
        ),
    )(feats, coors, coorsT, W_q, W_kvT, W_out, b_out.reshape(1, DIM), invf,
      W_c1, b_c1.reshape(1, CH), W_c2, b_c2.reshape(1, H),
      W_gate, b_gate.reshape(1, H), coors_combine.reshape(1, H),
      coors_norm_scale.reshape(1, 1))
    return out, cout


# speedup vs baseline: 1.4642x; 1.4642x over previous
import functools

import jax
import jax.numpy as jnp
from jax.experimental import pallas as pl
from jax.experimental.pallas import tpu as pltpu

B = 2
N = 256
DIM = 512
H = 8
DH = 64
CH = 16
REL_DIST_SCALE = 100.0
REL_DIST_CUTOFF = 5000.0
ROTARY_THETA = 10000.0
SCALE = DH ** (-0.5)

BI = 32
NI = N // BI
HP = DH // 2
HD2 = DIM // 2

_PREC = jax.lax.Precision.HIGHEST


def _gelu(x):
    return x * (jax.lax.erf(x * (2.0 ** -0.5)) + 1.0) * 0.5


def _softmax_lanes(x):
    m = jnp.max(x, axis=-1, keepdims=True)
    e = jnp.exp(x - m)
    return e / jnp.sum(e, axis=-1, keepdims=True)


def _evenodd_rot(xT):
    x2 = xT.reshape(HD2, 2, N)
    te = x2[:, 0, :]
    to = x2[:, 1, :]
    x4 = xT.reshape(DIM // 4, 4, N)
    x0 = x4[:, 0, :].reshape(H, CH, N)
    x1 = x4[:, 1, :].reshape(H, CH, N)
    x2b = x4[:, 2, :].reshape(H, CH, N)
    x3 = x4[:, 3, :].reshape(H, CH, N)
    ra = jnp.concatenate([-x1, x0], axis=1).reshape(HD2, N)
    rb = jnp.concatenate([-x3, x2b], axis=1).reshape(HD2, N)
    return te, to, ra, rb


def _body(feats_ref, coors_ref, coorsT_ref, Wq_ref, WkvT_ref, Wout_ref,
          bout_ref, invf_ref, Wc1T_ref, bc1_ref, Wc2T_ref, bc2_ref,
          WgT_ref, bg_ref, comb_ref, cns_ref,
          out_ref, cout_ref,
          qs_ref, kTe_ref, kTo_ref, kA_ref, kB_ref,
          vTe_ref, vTo_ref, vA_ref, vB_ref):
    i = pl.program_id(1)

    @pl.when(i == 0)
    def _setup():
        f = feats_ref[0]
        qs_ref[...] = jnp.dot(f, Wq_ref[...],
                              preferred_element_type=jnp.float32,
                              precision=_PREC) * SCALE
        kvT = jnp.dot(WkvT_ref[...], f.T,
                      preferred_element_type=jnp.float32,
                      precision=_PREC)
        kTe_ref[...], kTo_ref[...], kA_ref[...], kB_ref[...] = (
            _evenodd_rot(kvT[:DIM]))
        vTe_ref[...], vTo_ref[...], vA_ref[...], vB_ref[...] = (
            _evenodd_rot(kvT[DIM:]))

    ci = coors_ref[0, pl.ds(i * BI, BI), :]
    cj = coorsT_ref[0]
    rel = [ci[:, c:c + 1] - cj[c:c + 1, :] for c in range(3)]
    sq = rel[0] * rel[0] + rel[1] * rel[1] + rel[2] * rel[2]
    dist = jnp.where(sq == 0.0, 0.0, jnp.sqrt(jnp.where(sq == 0.0, 1.0, sq)))
    rd = jnp.minimum(dist * REL_DIST_SCALE, REL_DIST_CUTOFF)
    f3 = rd[:, None, :] * invf_ref[...][None, :, :]
    cosf = jnp.cos(f3)
    sinf = jnp.sin(f3)

    q = qs_ref[pl.ds(i * BI, BI), :]
    qeo = q.reshape(BI, HD2, 2)
    qe = qeo[:, :, 0][:, :, None]
    qo = qeo[:, :, 1][:, :, None]
    c4 = (qe * kTe_ref[...][None]
          + qo * kTo_ref[...][None]).reshape(BI, H, HP, N)
    s4 = (qe * kA_ref[...][None]
          + qo * kB_ref[...][None]).reshape(BI, H, HP, N)
    t4 = c4 * cosf[:, None] + s4 * sinf[:, None]
    qk = jnp.sum(t4, axis=2)

    qkT = qk.transpose(1, 0, 2).reshape(H, BI * N)
    hmid = _gelu(jnp.dot(Wc1T_ref[...], qkT,
                         preferred_element_type=jnp.float32) + bc1_ref[...])
    cw = (jnp.dot(Wc2T_ref[...], hmid,
                  preferred_element_type=jnp.float32) + bc2_ref[...])
    gate = jnp.tanh(jnp.dot(WgT_ref[...], qkT,
                            preferred_element_type=jnp.float32) + bg_ref[...])
    coor_attn = _softmax_lanes(cw.reshape(H, BI, N))
    gate = gate.reshape(H, BI, N)
    wsum = jnp.sum(coor_attn * gate * comb_ref[...][:, :, None], axis=0)
    inv = cns_ref[0, 0] / jnp.maximum(dist, 1e-8)
    wr = wsum * inv
    cout_ref[0] = jnp.concatenate(
        [jnp.sum(wr * rel[c], axis=1, keepdims=True) for c in range(3)],
        axis=1)

    attn = _softmax_lanes(qk)
    ac4 = attn[:, :, None, :] * cosf[:, None]
    as4 = attn[:, :, None, :] * sinf[:, None]
    vTe4 = vTe_ref[...].reshape(H, HP, N)[None]
    vTo4 = vTo_ref[...].reshape(H, HP, N)[None]
    vA4 = vA_ref[...].reshape(H, HP, N)[None]
    vB4 = vB_ref[...].reshape(H, HP, N)[None]
    oe = jnp.sum(ac4 * vTe4 + as4 * vA4, axis=3)
    oo = jnp.sum(ac4 * vTo4 + as4 * vB4, axis=3)
    out_cat = jnp.stack([oe, oo], axis=-1).reshape(BI, DIM)
    out_ref[0, pl.ds(i * BI, BI), :] = out_cat

    @pl.when(i == NI - 1)
    def _project():
        out_ref[0] = jnp.dot(out_ref[0], Wout_ref[...],
                             preferred_element_type=jnp.float32,
                             precision=_PREC) + bout_ref[...]


@jax.jit
def kernel(feats, coors, W_qkv, W_out, b_out, W_c1, b_c1, W_c2, b_c2,
           W_gate, b_gate, coors_norm_scale, coors_combine):
    W_q = W_qkv[:, :DIM]
    W_kvT = W_qkv[:, DIM:].T
    coorsT = coors.transpose(0, 2, 1)
    invf = (1.0 / (ROTARY_THETA **
                   (jnp.arange(0, DH, 2, dtype=jnp.float32) / DH)))
    invf = jnp.tile(invf.reshape(HP, 1), (1, N))
    vfull = lambda shape, imap: pl.BlockSpec(shape, imap)
    const2 = lambda b, i: (0, 0)
    out, cout = pl.pallas_call(
        _body,
        grid=(B, NI),
        in_specs=[
            vfull((1, N, DIM), lambda b, i: (b, 0, 0)),
            vfull((1, N, 3), lambda b, i: (b, 0, 0)),
            vfull((1, 3, N), lambda b, i: (b, 0, 0)),
            vfull((DIM, DIM), const2),
            vfull((2 * DIM, DIM), const2),
            vfull((DIM, DIM), const2),
            vfull((1, DIM), const2),
            vfull((HP, N), const2),
            vfull((CH, H), const2),
            vfull((CH, 1), const2),
            vfull((H, CH), const2),
            vfull((H, 1), const2),
            vfull((H, H), const2),
            vfull((H, 1), const2),
            vfull((H, 1), const2),
            vfull((1, 1), const2),
        ],
        out_specs=[
            pl.BlockSpec((1, N, DIM), lambda b, i: (b, 0, 0)),
            pl.BlockSpec((1, BI, 3), lambda b, i: (b, i, 0)),
        ],
        out_shape=[
            jax.ShapeDtypeStruct((B, N, DIM), jnp.float32),
            jax.ShapeDtypeStruct((B, N, 3), jnp.float32),
        ],
        scratch_shapes=[
            pltpu.VMEM((N, DIM), jnp.float32),
            pltpu.VMEM((HD2, N), jnp.float32),
            pltpu.VMEM((HD2, N), jnp.float32),
            pltpu.VMEM((HD2, N), jnp.float32),
            pltpu.VMEM((HD2, N), jnp.float32),
            pltpu.VMEM((HD2, N), jnp.float32),
            pltpu.VMEM((HD2, N), jnp.float32),
            pltpu.VMEM((HD2, N), jnp.float32),
            pltpu.VMEM((HD2, N), jnp.float32),
        ],
        compiler_params=pltpu.CompilerParams(
            dimension_semantics=("arbitrary", "arbitrary"),
        ),
    )(feats, coors, coorsT, W_q, W_kvT, W_out, b_out.reshape(1, DIM), invf,
      W_c1.T, b_c1.reshape(CH, 1), W_c2.T, b_c2.reshape(H, 1),
      W_gate.T, b_gate.reshape(H, 1), coors_combine.reshape(H, 1),
      coors_norm_scale.reshape(1, 1))
    return out, cout
